# Initial kernel scaffold; baseline (speedup 1.0000x reference)
import jax
import jax.numpy as jnp
from jax.experimental import pallas as pl


def kernel(x):
    raise NotImplementedError("write your pallas kernel here")


# speedup vs baseline: 2.7347x; 2.7347x over previous
import functools

import jax
import jax.numpy as jnp
from jax import lax
from jax.experimental import pallas as pl
from jax.experimental.pallas import tpu as pltpu
from jax.experimental.pallas import tpu_sc as plsc

B, CH = 32, 768
H, W = 32, 32
GH, GW = 16, 16
NG = GH * GW
MERGE = 128
NKEEP = 512
NOUT = MERGE + NKEEP
ROWS_PER_CHUNK = 64
_HIGHEST = jax.lax.Precision.HIGHEST


def _dotg(a, b, ca, cb):
    return lax.dot_general(
        a, b, (((ca,), (cb,)), ((), ())),
        precision=_HIGHEST, preferred_element_type=jnp.float32)


def _red_c(d):
    s0 = jnp.sum(d[:, 0:128] + d[:, 128:256], axis=1, keepdims=True)
    s1 = jnp.sum(d[:, 256:384] + d[:, 384:512], axis=1, keepdims=True)
    s2 = jnp.sum(d[:, 512:640] + d[:, 640:768], axis=1, keepdims=True)
    return (s0 + s1) + s2


def _roll_sub(a, shift):
    return pltpu.roll(a, shift % a.shape[0], 0)


def _stage_a_body(xt_ref, score_ref, pool_ref):
    a0 = xt_ref[0, 0:32, :]
    a1 = xt_ref[0, 32:64, :]

    w_iota = lax.broadcasted_iota(jnp.int32, (W, 1), 0)
    w_odd = (w_iota & 1) == 1
    x0 = jnp.where(w_odd, _roll_sub(a0, 1), a0)

    dot0 = _red_c(x0 * a0)
    dot1 = _red_c(x0 * a1)
    n2_0 = _red_c(a0 * a0)
    n2_1 = _red_c(a1 * a1)
    n1_sq = _red_c(x0 * x0)

    n1s = jnp.sqrt(n1_sq)
    eps = jnp.float32(1e-8)
    cos0 = dot0 / jnp.maximum(n1s * jnp.sqrt(n2_0), eps)
    cos1 = dot1 / jnp.maximum(n1s * jnp.sqrt(n2_1), eps)

    th = cos0 + cos1
    sc = (th + _roll_sub(th, -1)) * jnp.float32(0.25)

    li = lax.broadcasted_iota(jnp.int32, (GW, W), 1)
    si = lax.broadcasted_iota(jnp.int32, (GW, W), 0)
    comp = (li == 2 * si).astype(jnp.float32)
    score_ref[0, :, :] = _dotg(comp, sc, 1, 0)

    psum = (a0 + _roll_sub(a0, -1)) + (a1 + _roll_sub(a1, -1))
    pool_ref[0, :, :] = _dotg(comp, psum, 1, 0) * jnp.float32(0.25)


def _stage_b_body(score_ref, aligned_ref, gidx_ref):
    b = pl.program_id(0)
    s_col = score_ref[0]
    g_sub = lax.broadcasted_iota(jnp.int32, (NG, NG), 0)
    g_lane = lax.broadcasted_iota(jnp.int32, (NG, NG), 1)
    ident = (g_sub == g_lane).astype(jnp.float32)
    s_row = _dotg(s_col, ident, 0, 0)

    less = s_row < s_col
    eq = s_row == s_col
    m = jnp.where(less | (eq & (g_lane < g_sub)), 1, 0)
    rank_col = jnp.sum(m, axis=1, keepdims=True)

    r_lane = lax.broadcasted_iota(jnp.int32, (NG, NG), 1)
    oht = (rank_col == r_lane).astype(jnp.float32)

    g_col = lax.broadcasted_iota(jnp.int32, (NG, 1), 0)
    tok0_col = ((g_col >> 4) << 6) + ((g_col & 15) << 1)
    order_row = _dotg(g_col.astype(jnp.float32), oht, 0, 0)
    tok0_row = _dotg(tok0_col.astype(jnp.float32), oht, 0, 0)

    order_m = order_row[:, 0:MERGE].astype(jnp.int32)
    tok0_m = tok0_row[:, 0:MERGE].astype(jnp.int32)
    tok0_k = tok0_row[:, MERGE:NG].astype(jnp.int32)

    aligned_ref[0, 0, 0:128] = tok0_m[0]
    gidx_ref[0, 0, 0:128] = order_m[0] + b * NG
    for p, off in enumerate((0, 1, 32, 33)):
        tok_p = tok0_k[0] + off
        aligned_ref[0, 0, 128 + 128 * p:256 + 128 * p] = tok_p
        gidx_ref[0, 0, 128 + 128 * p:256 + 128 * p] = tok_p + b * (H * W)


def _make_gather_kernel():
    mesh = plsc.VectorSubcoreMesh(core_axis_name="c", subcore_axis_name="s")

    @functools.partial(
        pl.kernel,
        mesh=mesh,
        out_type=jax.ShapeDtypeStruct((B * NOUT, CH), jnp.float32),
        scratch_types=[
            pltpu.VMEM((NOUT,), jnp.int32),
            pltpu.VMEM((ROWS_PER_CHUNK, CH), jnp.float32),
            pltpu.VMEM((ROWS_PER_CHUNK, CH), jnp.float32),
            pltpu.SemaphoreType.DMA,
            pltpu.SemaphoreType.DMA,
        ],
    )
    def gather_kernel(xt_hbm, pool_hbm, gidx_hbm, out_hbm, idx_v, buf0, buf1, sem0, sem1):
        wid = lax.axis_index("s") * 2 + lax.axis_index("c")
        pltpu.sync_copy(gidx_hbm.at[wid], idx_v)
        bufs = (buf0, buf1)
        sems = (sem0, sem1)
        nchunks = NOUT // ROWS_PER_CHUNK

        def start(k):
            table = pool_hbm if k < 2 else xt_hbm
            idx_sl = idx_v.at[pl.ds(k * ROWS_PER_CHUNK, ROWS_PER_CHUNK)]
            return pltpu.async_copy(table.at[idx_sl], bufs[k % 2], sems[k % 2])

        cp = start(0)
        for k in range(nchunks):
            nxt = start(k + 1) if k + 1 < nchunks else None
            cp.wait()
            pltpu.sync_copy(
                bufs[k % 2],
                out_hbm.at[pl.ds(wid * NOUT + k * ROWS_PER_CHUNK, ROWS_PER_CHUNK)])
            cp = nxt

    return gather_kernel


def kernel(x):
    b, c, h, w = x.shape
    xt = jnp.transpose(x, (0, 2, 3, 1)).reshape(b, h * w, c)

    score_col, pool = pl.pallas_call(
        _stage_a_body,
        grid=(b, GH),
        in_specs=[pl.BlockSpec((1, 2 * W, c), lambda i, j: (i, j, 0))],
        out_specs=[
            pl.BlockSpec((1, GW, 1), lambda i, j: (i, j, 0)),
            pl.BlockSpec((1, GW, c), lambda i, j: (i, j, 0)),
        ],
        out_shape=[
            jax.ShapeDtypeStruct((b, NG, 1), jnp.float32),
            jax.ShapeDtypeStruct((b, NG, c), jnp.float32),
        ],
    )(xt)

    aligned, gidx = pl.pallas_call(
        _stage_b_body,
        grid=(b,),
        in_specs=[pl.BlockSpec((1, NG, 1), lambda i: (i, 0, 0))],
        out_specs=[
            pl.BlockSpec((1, 1, NOUT), lambda i: (i, 0, 0)),
            pl.BlockSpec((1, 1, NOUT), lambda i: (i, 0, 0)),
        ],
        out_shape=[
            jax.ShapeDtypeStruct((b, 1, NOUT), jnp.int32),
            jax.ShapeDtypeStruct((b, 1, NOUT), jnp.int32),
        ],
    )(score_col)
    aligned = aligned.reshape(b, NOUT)
    gidx = gidx.reshape(b, NOUT)

    out_flat = _make_gather_kernel()(
        xt.reshape(b * h * w, c), pool.reshape(b * NG, c), gidx)
    return out_flat.reshape(b, NOUT, c), aligned


# speedup vs baseline: 7.5948x; 2.7772x over previous
import functools

import jax
import jax.numpy as jnp
from jax import lax
from jax.experimental import pallas as pl
from jax.experimental.pallas import tpu as pltpu
from jax.experimental.pallas import tpu_sc as plsc

B, CH = 32, 768
H, W = 32, 32
GH, GW = 16, 16
NG = GH * GW
MERGE = 128
NKEEP = 512
NOUT = MERGE + NKEEP
ROWS_PER_CHUNK = 64
_HIGHEST = jax.lax.Precision.HIGHEST


def _dotg(a, b, ca, cb):
    return lax.dot_general(
        a, b, (((ca,), (cb,)), ((), ())),
        precision=_HIGHEST, preferred_element_type=jnp.float32)


def _red_c(d):
    s0 = jnp.sum(d[:, 0:128] + d[:, 128:256], axis=1, keepdims=True)
    s1 = jnp.sum(d[:, 256:384] + d[:, 384:512], axis=1, keepdims=True)
    s2 = jnp.sum(d[:, 512:640] + d[:, 640:768], axis=1, keepdims=True)
    return (s0 + s1) + s2


def _roll_sub(a, shift):
    return pltpu.roll(a, shift % a.shape[0], 0)


def _stage_a_body(xt_ref, score_ref, pool_ref):
    w_iota = lax.broadcasted_iota(jnp.int32, (W, 1), 0)
    w_odd = (w_iota & 1) == 1
    li = lax.broadcasted_iota(jnp.int32, (GW, W), 1)
    si = lax.broadcasted_iota(jnp.int32, (GW, W), 0)
    comp = (li == 2 * si).astype(jnp.float32)
    eps = jnp.float32(1e-8)

    for j in range(GH):
        a0 = xt_ref[0, 64 * j:64 * j + 32, :]
        a1 = xt_ref[0, 64 * j + 32:64 * j + 64, :]
        x0 = jnp.where(w_odd, _roll_sub(a0, 1), a0)

        dot0 = _red_c(x0 * a0)
        dot1 = _red_c(x0 * a1)
        n2_0 = _red_c(a0 * a0)
        n2_1 = _red_c(a1 * a1)
        n1_sq = _red_c(x0 * x0)

        n1s = jnp.sqrt(n1_sq)
        cos0 = dot0 / jnp.maximum(n1s * jnp.sqrt(n2_0), eps)
        cos1 = dot1 / jnp.maximum(n1s * jnp.sqrt(n2_1), eps)

        th = cos0 + cos1
        sc = (th + _roll_sub(th, -1)) * jnp.float32(0.25)

        score_ref[0, GW * j:GW * (j + 1), :] = _dotg(comp, sc, 1, 0)

        psum = (a0 + _roll_sub(a0, -1)) + (a1 + _roll_sub(a1, -1))
        pool_ref[0, GW * j:GW * (j + 1), :] = lax.dot_general(
            comp, psum, (((1,), (0,)), ((), ())),
            preferred_element_type=jnp.float32) * jnp.float32(0.25)


def _stage_b_body(score_ref, aligned_ref, gidx_ref):
    b = pl.program_id(0)
    s_col = score_ref[0]
    g_sub = lax.broadcasted_iota(jnp.int32, (NG, NG), 0)
    g_lane = lax.broadcasted_iota(jnp.int32, (NG, NG), 1)
    ident = (g_sub == g_lane).astype(jnp.float32)
    s_row = _dotg(s_col, ident, 0, 0)

    less = s_row < s_col
    eq = s_row == s_col
    m = jnp.where(less | (eq & (g_lane < g_sub)), 1, 0)
    rank_col = jnp.sum(m, axis=1, keepdims=True)

    r_lane = lax.broadcasted_iota(jnp.int32, (NG, NG), 1)
    oht = (rank_col == r_lane).astype(jnp.float32)

    g_col = lax.broadcasted_iota(jnp.int32, (NG, 1), 0)
    tok0_col = ((g_col >> 4) << 6) + ((g_col & 15) << 1)
    order_row = _dotg(g_col.astype(jnp.float32), oht, 0, 0)
    tok0_row = _dotg(tok0_col.astype(jnp.float32), oht, 0, 0)

    order_m = order_row[:, 0:MERGE].astype(jnp.int32)
    tok0_m = tok0_row[:, 0:MERGE].astype(jnp.int32)
    tok0_k = tok0_row[:, MERGE:NG].astype(jnp.int32)

    aligned_ref[0, 0, 0:128] = tok0_m[0]
    gidx_ref[0, 0, 0:128] = order_m[0] + b * NG
    for p, off in enumerate((0, 1, 32, 33)):
        tok_p = tok0_k[0] + off
        aligned_ref[0, 0, 128 + 128 * p:256 + 128 * p] = tok_p
        gidx_ref[0, 0, 128 + 128 * p:256 + 128 * p] = tok_p + b * (H * W)


def _make_gather_kernel():
    mesh = plsc.VectorSubcoreMesh(core_axis_name="c", subcore_axis_name="s")

    @functools.partial(
        pl.kernel,
        mesh=mesh,
        out_type=jax.ShapeDtypeStruct((B * NOUT, CH), jnp.float32),
        scratch_types=[
            pltpu.VMEM((NOUT,), jnp.int32),
            pltpu.VMEM((ROWS_PER_CHUNK, CH), jnp.float32),
            pltpu.VMEM((ROWS_PER_CHUNK, CH), jnp.float32),
            pltpu.SemaphoreType.DMA,
            pltpu.SemaphoreType.DMA,
        ],
    )
    def gather_kernel(xt_hbm, pool_hbm, gidx_hbm, out_hbm, idx_v, buf0, buf1, sem0, sem1):
        wid = lax.axis_index("s") * 2 + lax.axis_index("c")
        pltpu.sync_copy(gidx_hbm.at[wid], idx_v)
        bufs = (buf0, buf1)
        sems = (sem0, sem1)
        nchunks = NOUT // ROWS_PER_CHUNK

        def start(k):
            table = pool_hbm if k < 2 else xt_hbm
            idx_sl = idx_v.at[pl.ds(k * ROWS_PER_CHUNK, ROWS_PER_CHUNK)]
            return pltpu.async_copy(table.at[idx_sl], bufs[k % 2], sems[k % 2])

        cp = start(0)
        for k in range(nchunks):
            nxt = start(k + 1) if k + 1 < nchunks else None
            cp.wait()
            pltpu.sync_copy(
                bufs[k % 2],
                out_hbm.at[pl.ds(wid * NOUT + k * ROWS_PER_CHUNK, ROWS_PER_CHUNK)])
            cp = nxt

    return gather_kernel


def kernel(x):
    b, c, h, w = x.shape
    xt = jnp.transpose(x, (0, 2, 3, 1)).reshape(b, h * w, c)

    score_col, pool = pl.pallas_call(
        _stage_a_body,
        grid=(b,),
        in_specs=[pl.BlockSpec((1, h * w, c), lambda i: (i, 0, 0))],
        out_specs=[
            pl.BlockSpec((1, NG, 1), lambda i: (i, 0, 0)),
            pl.BlockSpec((1, NG, c), lambda i: (i, 0, 0)),
        ],
        out_shape=[
            jax.ShapeDtypeStruct((b, NG, 1), jnp.float32),
            jax.ShapeDtypeStruct((b, NG, c), jnp.float32),
        ],
    )(xt)

    aligned, gidx = pl.pallas_call(
        _stage_b_body,
        grid=(b,),
        in_specs=[pl.BlockSpec((1, NG, 1), lambda i: (i, 0, 0))],
        out_specs=[
            pl.BlockSpec((1, 1, NOUT), lambda i: (i, 0, 0)),
            pl.BlockSpec((1, 1, NOUT), lambda i: (i, 0, 0)),
        ],
        out_shape=[
            jax.ShapeDtypeStruct((b, 1, NOUT), jnp.int32),
            jax.ShapeDtypeStruct((b, 1, NOUT), jnp.int32),
        ],
    )(score_col)
    aligned = aligned.reshape(b, NOUT)
    gidx = gidx.reshape(b, NOUT)

    out_flat = _make_gather_kernel()(
        xt.reshape(b * h * w, c), pool.reshape(b * NG, c), gidx)
    return out_flat.reshape(b, NOUT, c), aligned
